# baseline (speedup 1.0000x reference)
import functools

import jax
import jax.numpy as jnp
from jax import lax
from jax.experimental import pallas as pl
from jax.experimental.pallas import tpu as pltpu
from jax.experimental.pallas import tpu_sc as plsc

SAMPLE_B = 1024
WRITE_B = 4096
D = 3 * 32 * 32

NC, NS = 2, 16
NW = NC * NS
ROWS_PER = SAMPLE_B // NW
CHUNK = 16
NCHUNK = ROWS_PER // CHUNK
PAD = 8


def _winner_body(idx_ref, s_ref, w_ref):
    ix = idx_ref[...]
    s = s_ref[...].reshape(1, 128)
    eq = ix == s
    j = lax.broadcasted_iota(jnp.int32, (WRITE_B, 128), 0)
    cand = jnp.where(eq, j, -1)
    w_ref[...] = jnp.max(cand, axis=0, keepdims=True).reshape(1, 1, 128)


def _winner_tc(idx, sample_idx):
    idx2 = idx.reshape(WRITE_B, 1)
    s3 = sample_idx.reshape(SAMPLE_B // 128, 1, 128)
    grid = SAMPLE_B // 128
    w = pl.pallas_call(
        _winner_body,
        grid=(grid,),
        in_specs=[
            pl.BlockSpec((WRITE_B, 1), lambda i: (0, 0)),
            pl.BlockSpec((1, 1, 128), lambda i: (i, 0, 0)),
        ],
        out_specs=pl.BlockSpec((1, 1, 128), lambda i: (i, 0, 0)),
        out_shape=jax.ShapeDtypeStruct((SAMPLE_B // 128, 1, 128), jnp.int32),
    )(idx2, s3)
    return w.reshape(SAMPLE_B)


def _merge_body(sidx_sref, win_sref, mem_ref, val_ref, out_ref):
    i = pl.program_id(0)
    w = win_sref[i]

    @pl.when(w >= 0)
    def _():
        out_ref[...] = val_ref[...]

    @pl.when(w < 0)
    def _():
        out_ref[...] = mem_ref[...]


def _merge_tc(mem, val, sample_idx, winner):
    row = mem.shape[1:]
    blk = (1,) + row

    grid_spec = pltpu.PrefetchScalarGridSpec(
        num_scalar_prefetch=2,
        grid=(SAMPLE_B,),
        in_specs=[
            pl.BlockSpec(blk, lambda i, s, w: (jnp.where(w[i] >= 0, 0, s[i]), 0, 0, 0)),
            pl.BlockSpec(blk, lambda i, s, w: (jnp.where(w[i] >= 0, w[i], 0), 0, 0, 0)),
        ],
        out_specs=pl.BlockSpec(blk, lambda i, s, w: (i, 0, 0, 0)),
    )
    return pl.pallas_call(
        _merge_body,
        grid_spec=grid_spec,
        out_shape=jax.ShapeDtypeStruct((SAMPLE_B,) + row, jnp.float32),
    )(sample_idx, winner, mem, val)


def _sc_gather(mem, val, sample_idx, winner):
    mesh = plsc.VectorSubcoreMesh(core_axis_name="c", subcore_axis_name="s")
    row_shape = mem.shape[1:]

    @functools.partial(
        pl.kernel,
        mesh=mesh,
        out_type=jax.ShapeDtypeStruct((SAMPLE_B + PAD,) + row_shape, jnp.float32),
        scratch_types=[
            pltpu.VMEM((ROWS_PER,), jnp.int32),
            pltpu.VMEM((ROWS_PER,), jnp.int32),
            pltpu.VMEM((CHUNK,) + row_shape, jnp.float32),
            pltpu.VMEM((CHUNK,) + row_shape, jnp.float32),
            pltpu.SemaphoreType.DMA,
            pltpu.SemaphoreType.DMA,
        ],
    )
    def k(mem_hbm, val_hbm, sidx_hbm, win_hbm, out_hbm,
          sidx_v, win_v, bufA, bufB, semA, semB):
        wid = lax.axis_index("s") * NC + lax.axis_index("c")
        base = wid * ROWS_PER

        pltpu.sync_copy(sidx_hbm.at[pl.ds(base, ROWS_PER)], sidx_v)
        pltpu.sync_copy(win_hbm.at[pl.ds(base, ROWS_PER)], win_v)

        dump = SAMPLE_B + (wid % PAD)
        lane = lax.broadcasted_iota(jnp.int32, (CHUNK,), 0)
        for c in range(NCHUNK):
            sidx = sidx_v[pl.ds(c * CHUNK, CHUNK)]
            w = win_v[pl.ds(c * CHUNK, CHUNK)]
            vsrc = jnp.maximum(w, 0)
            vdst = jnp.where(w >= 0, base + c * CHUNK + lane, dump)

            pltpu.async_copy(mem_hbm.at[sidx], bufA, semA).wait()
            pltpu.sync_copy(bufA, out_hbm.at[pl.ds(base + c * CHUNK, CHUNK)])
            pltpu.async_copy(val_hbm.at[vsrc], bufB, semB).wait()
            pltpu.async_copy(bufB, out_hbm.at[vdst], semB).wait()

    return k(mem, val, sample_idx, winner)


def kernel(mem, idx, val, sample_idx):
    winner = _winner_tc(idx, sample_idx)
    return _merge_tc(mem, val, sample_idx, winner)
